# baseline (speedup 1.0000x reference)
import functools

import numpy as np
import jax
import jax.numpy as jnp
from jax import lax
from jax.experimental import pallas as pl
from jax.experimental.pallas import tpu as pltpu
from jax.experimental.pallas import tpu_sc as plsc

B = 128
N = 784
D = 32
M = 256
TEMP = 1.0
L = 16
PAD = 784
NV = PAD // L
KT = N // 8
HALF_K = N // 2


_noise_cache = []


def _noise():
    if not _noise_cache:
        with jax.ensure_compile_time_eval():
            key = jax.random.key(42)
            kn1, kn2 = jax.random.split(key)
            n1 = -jnp.log(-jnp.log(
                jax.random.uniform(kn1, (B, N), minval=1e-6, maxval=1.0)))
            n2 = -jnp.log(-jnp.log(
                jax.random.uniform(kn2, (B, N), minval=1e-6, maxval=1.0)))
        _noise_cache.append((np.asarray(n1), np.asarray(n2)))
    return _noise_cache[0]


def _count(ref, thr_vec, strict):
    def blk(bi, acc):
        for j in range(7):
            x = ref[pl.ds((bi * 7 + j) * L, L)]
            m = (x > thr_vec) if strict else (x >= thr_vec)
            acc = acc + m.astype(jnp.int32)
        return acc
    acc = lax.fori_loop(0, NV // 7, blk, jnp.zeros((L,), jnp.int32))
    return jnp.sum(acc)


def _kth_value(ref, k):
    def bit_body(t, prefix):
        b = jnp.int32(29) - t
        trial = prefix | (jnp.int32(1) << b)
        cnt = _count(ref, jnp.broadcast_to(trial, (L,)), strict=False)
        return jnp.where(cnt >= k, trial, prefix)

    prefix = lax.fori_loop(0, 30, bit_body, jnp.int32(0))
    return jnp.broadcast_to(prefix, (L,))


def _select(ref, v_vec, k, out_ref, lane_iota):
    n_gt = _count(ref, v_vec, strict=True)
    need = k - n_gt

    def body(v, carry):
        off, eqacc = carry
        x = ref[pl.ds(v * L, L)]
        gt = x > v_vec
        eq = x == v_vec
        inc = eq.astype(jnp.int32)
        cse = plsc.cumsum(inc)
        sel = gt | (eq & ((eqacc + cse) <= need))
        selinc = sel.astype(jnp.int32)
        css = plsc.cumsum(selinc)
        dst = jnp.where(sel, off + css - 1, 0)
        plsc.store_scatter(out_ref, [dst], lane_iota + v * L, mask=sel)
        return off + jnp.sum(selinc), eqacc + jnp.sum(inc)
    lax.fori_loop(0, NV, body, (jnp.int32(0), jnp.int32(0)))


_call_cache = []


def _get_call():
    if _call_cache:
        return _call_cache[0]

    info = plsc.get_sparse_core_info()
    nc, ns = info.num_cores, info.num_subcores
    nw = nc * ns
    rows_per_w = B // nw
    half_f32 = (KT // 2) * 2048
    row_f32 = M * N
    mesh = plsc.VectorSubcoreMesh(core_axis_name="c", subcore_axis_name="s")

    @functools.partial(
        pl.kernel,
        out_type=jax.ShapeDtypeStruct((B * M * N,), jnp.float32),
        mesh=mesh,
        compiler_params=pltpu.CompilerParams(needs_layout_passes=False),
        scratch_types=[
            pltpu.VMEM((PAD,), jnp.int32),
            pltpu.VMEM((PAD,), jnp.int32),
            pltpu.VMEM((rows_per_w * (D + L),), jnp.int32),
            pltpu.VMEM((rows_per_w * (M + L),), jnp.int32),
            pltpu.VMEM(((KT // 2) * 2048,), jnp.float32),
            pltpu.VMEM((M,), jnp.int32),
            pltpu.VMEM((L,), jnp.float32),
            pltpu.VMEM((M * D,), jnp.int32),
            pltpu.VMEM((M * D,), jnp.int32),
            pltpu.SemaphoreType.DMA,
        ],
    )
    def sc_call(probs1, probs2, cvec, out, p1, p2, kidx, ridx,
                buf, mparts, cbuf, plog0, plog1, osem):
        wid = lax.axis_index("s") * nc + lax.axis_index("c")
        gi0 = wid * rows_per_w
        lane_iota = lax.iota(jnp.int32, L)
        zf = jnp.zeros((L,), jnp.float32)

        pltpu.sync_copy(cvec, cbuf)

        for g in range(M // L):
            mv = lane_iota + (g * L)
            mparts[pl.ds(g * L, L)] = ((mv >> 7) << 10) + (mv & 127)

        def zbody(z, _):
            for j in range(8):
                buf[pl.ds(z * 128 + j * L, L)] = zf
            return 0
        lax.fori_loop(0, half_f32 // 128, zbody, 0)

        def piece1(row):
            gi = gi0 + row
            pltpu.sync_copy(probs1.at[pl.ds(gi * N, N)], p1)
            v1 = _kth_value(p1, D)
            _select(p1, v1, D, kidx.at[pl.ds(row * (D + L), D + L)], lane_iota)

        def piece2(row):
            gi = gi0 + row
            pltpu.sync_copy(probs2.at[pl.ds(gi * N, N)], p2)
            v2 = _kth_value(p2, M)
            _select(p2, v2, M, ridx.at[pl.ds(row * (M + L), M + L)], lane_iota)

        piece1(0)
        piece2(0)

        cval = cbuf[...]

        def pairpass(row, plog):
            def dbody(d, _):
                kd = kidx[pl.ds(row * (D + L) + d, L)][0]
                for g in range(M // L):
                    rv = ridx[pl.ds(row * (M + L) + g * L, L)]
                    mpart = mparts[pl.ds(g * L, L)]
                    k = rv + kd
                    k = jnp.where(k >= N, k - N, k)
                    plog[pl.ds((d * (M // L) + g) * L, L)] = (
                        ((k >> 3) << 11) + ((k & 7) << 7) + mpart)
                return 0
            lax.fori_loop(0, D, dbody, 0)

        def replay(plog, cbase, value_vec):
            def cbody(v, _):
                for j in range(8):
                    pv = plog[pl.ds((v * 8 + j) * L, L)]
                    u = pv - cbase
                    msk = (u >= 0) & (u < half_f32)
                    u = jnp.where(msk, u, 0)
                    plsc.store_scatter(buf, [u], value_vec, mask=msk)
                return 0
            lax.fori_loop(0, M * D // L // 8, cbody, 0)

        pairpass(0, plog0)
        plogs = (plog0, plog1)
        nsteps = rows_per_w * 2
        for t in range(nsteps):
            row, half = t // 2, t % 2
            plog = plogs[row % 2]
            cbase = half * half_f32
            replay(plog, cbase, cval)
            base = (gi0 + row) * row_f32 + cbase
            piece = half_f32 // 2
            cp0 = pltpu.make_async_copy(
                buf.at[pl.ds(0, piece)], out.at[pl.ds(base, piece)], osem)
            cp1 = pltpu.make_async_copy(
                buf.at[pl.ds(piece, piece)],
                out.at[pl.ds(base + piece, piece)], osem)
            cp0.start()
            cp1.start()
            if row + 1 < rows_per_w:
                if half == 0:
                    piece1(row + 1)
                else:
                    piece2(row + 1)
                    pairpass(row + 1, plogs[(row + 1) % 2])
            cp0.wait()
            cp1.wait()
            if t != nsteps - 1:
                replay(plog, cbase, zf)

    _call_cache.append(sc_call)
    return sc_call


def kernel(scalar, kernel_param, mask_param, b):
    n1, n2 = (jnp.asarray(a) for a in _noise())
    logits = jnp.tile(kernel_param, (B, 1))
    probs1 = jax.nn.softmax((logits + n1 / 1000) / TEMP, axis=1)
    logits_m = jnp.tile(mask_param, (B, 1))
    probs2 = jax.nn.softmax((logits_m + n2 / 1000) / TEMP, axis=1)
    s = jnp.maximum(jnp.float32(0.01), scalar)
    cval = (jnp.ones((1,), jnp.float32) / np.sqrt(N)) * s
    cvec = jnp.broadcast_to(cval, (L,))
    bits1 = lax.bitcast_convert_type(probs1, jnp.int32).reshape(-1)
    bits2 = lax.bitcast_convert_type(probs2, jnp.int32).reshape(-1)
    flat = _get_call()(bits1, bits2, cvec)
    return (flat.reshape(B, N // 8, M // 128, 8, 128)
            .transpose(0, 2, 4, 1, 3)
            .reshape(B, M, N))


# speedup vs baseline: 1.0502x; 1.0502x over previous
import functools

import numpy as np
import jax
import jax.numpy as jnp
from jax import lax
from jax.experimental import pallas as pl
from jax.experimental.pallas import tpu as pltpu
from jax.experimental.pallas import tpu_sc as plsc

B = 128
N = 784
D = 32
M = 256
TEMP = 1.0
L = 16
PAD = 784
NV = PAD // L
KT = N // 8
HALF_K = N // 2


_noise_cache = []


def _noise():
    if not _noise_cache:
        with jax.ensure_compile_time_eval():
            key = jax.random.key(42)
            kn1, kn2 = jax.random.split(key)
            n1 = -jnp.log(-jnp.log(
                jax.random.uniform(kn1, (B, N), minval=1e-6, maxval=1.0)))
            n2 = -jnp.log(-jnp.log(
                jax.random.uniform(kn2, (B, N), minval=1e-6, maxval=1.0)))
        _noise_cache.append((np.asarray(n1), np.asarray(n2)))
    return _noise_cache[0]


def _count(ref, thr_vec, strict):
    def blk(bi, acc):
        for j in range(7):
            x = ref[pl.ds((bi * 7 + j) * L, L)]
            m = (x > thr_vec) if strict else (x >= thr_vec)
            acc = acc + m.astype(jnp.int32)
        return acc
    acc = lax.fori_loop(0, NV // 7, blk, jnp.zeros((L,), jnp.int32))
    return jnp.sum(acc)


def _kth_value(ref, k):
    def bit_body(t, prefix):
        b = jnp.int32(29) - t
        trial = prefix | (jnp.int32(1) << b)
        cnt = _count(ref, jnp.broadcast_to(trial, (L,)), strict=False)
        return jnp.where(cnt >= k, trial, prefix)

    prefix = lax.fori_loop(0, 30, bit_body, jnp.int32(0))
    return jnp.broadcast_to(prefix, (L,))


def _select(ref, v_vec, k, out_ref, lane_iota):
    n_gt = _count(ref, v_vec, strict=True)
    need = k - n_gt

    def body(v, carry):
        off, eqacc = carry
        x = ref[pl.ds(v * L, L)]
        gt = x > v_vec
        eq = x == v_vec
        inc = eq.astype(jnp.int32)
        cse = plsc.cumsum(inc)
        sel = gt | (eq & ((eqacc + cse) <= need))
        selinc = sel.astype(jnp.int32)
        css = plsc.cumsum(selinc)
        dst = jnp.where(sel, off + css - 1, 0)
        plsc.store_scatter(out_ref, [dst], lane_iota + v * L, mask=sel)
        return off + jnp.sum(selinc), eqacc + jnp.sum(inc)
    lax.fori_loop(0, NV, body, (jnp.int32(0), jnp.int32(0)))


_call_cache = []


def _get_call():
    if _call_cache:
        return _call_cache[0]

    info = plsc.get_sparse_core_info()
    nc, ns = info.num_cores, info.num_subcores
    nw = nc * ns
    rows_per_w = B // nw
    half_f32 = (KT // 2) * 2048
    row_f32 = M * N
    mesh = plsc.VectorSubcoreMesh(core_axis_name="c", subcore_axis_name="s")

    @functools.partial(
        pl.kernel,
        out_type=jax.ShapeDtypeStruct((B * M * N,), jnp.float32),
        mesh=mesh,
        compiler_params=pltpu.CompilerParams(needs_layout_passes=False),
        scratch_types=[
            pltpu.VMEM((PAD,), jnp.int32),
            pltpu.VMEM((PAD,), jnp.int32),
            pltpu.VMEM((rows_per_w * (D + L),), jnp.int32),
            pltpu.VMEM((rows_per_w * (M + L),), jnp.int32),
            pltpu.VMEM(((KT // 2) * 2048,), jnp.float32),
            pltpu.VMEM((M,), jnp.int32),
            pltpu.VMEM((L,), jnp.float32),
            pltpu.VMEM((M * D,), jnp.int32),
            pltpu.SemaphoreType.DMA,
        ],
    )
    def sc_call(probs1, probs2, cvec, out, p1, p2, kidx, ridx,
                buf, mparts, cbuf, plog, osem):
        wid = lax.axis_index("s") * nc + lax.axis_index("c")
        gi0 = wid * rows_per_w
        lane_iota = lax.iota(jnp.int32, L)
        zf = jnp.zeros((L,), jnp.float32)

        pltpu.sync_copy(cvec, cbuf)

        for g in range(M // L):
            mv = lane_iota + (g * L)
            mparts[pl.ds(g * L, L)] = ((mv >> 7) << 10) + (mv & 127)

        def zbody(z, _):
            for j in range(8):
                buf[pl.ds(z * 128 + j * L, L)] = zf
            return 0
        lax.fori_loop(0, half_f32 // 128, zbody, 0)

        def piece1(row):
            gi = gi0 + row
            pltpu.sync_copy(probs1.at[pl.ds(gi * N, N)], p1)
            v1 = _kth_value(p1, D)
            _select(p1, v1, D, kidx.at[pl.ds(row * (D + L), D + L)], lane_iota)

        def piece2(row):
            gi = gi0 + row
            pltpu.sync_copy(probs2.at[pl.ds(gi * N, N)], p2)
            v2 = _kth_value(p2, M)
            _select(p2, v2, M, ridx.at[pl.ds(row * (M + L), M + L)], lane_iota)

        piece1(0)
        piece2(0)

        cval = cbuf[...]

        def build(row, c0k):
            def dbody(d, _):
                kd = kidx[pl.ds(row * (D + L) + d, L)][0]
                for g in range(M // L):
                    rv = ridx[pl.ds(row * (M + L) + g * L, L)]
                    mpart = mparts[pl.ds(g * L, L)]
                    k = rv + kd
                    k = jnp.where(k >= N, k - N, k)
                    u = k - c0k
                    msk = (u >= 0) & (u < HALF_K)
                    poff = ((u >> 3) << 11) + ((u & 7) << 7) + mpart
                    poff = jnp.where(msk, poff, 0)
                    plsc.store_scatter(buf, [poff], cval, mask=msk)
                    plog[pl.ds((d * (M // L) + g) * L, L)] = poff
                return 0
            lax.fori_loop(0, D, dbody, 0)

        def clean():
            def cbody(v, _):
                for j in range(8):
                    pv = plog[pl.ds((v * 8 + j) * L, L)]
                    plsc.store_scatter(buf, [pv], zf)
                return 0
            lax.fori_loop(0, M * D // L // 8, cbody, 0)

        nsteps = rows_per_w * 2
        for t in range(nsteps):
            row, half = t // 2, t % 2
            c0k = half * HALF_K
            build(row, c0k)
            base = (gi0 + row) * row_f32 + half * half_f32
            piece = half_f32 // 2
            cp0 = pltpu.make_async_copy(
                buf.at[pl.ds(0, piece)], out.at[pl.ds(base, piece)], osem)
            cp1 = pltpu.make_async_copy(
                buf.at[pl.ds(piece, piece)],
                out.at[pl.ds(base + piece, piece)], osem)
            cp0.start()
            cp1.start()
            if row + 1 < rows_per_w:
                if half == 0:
                    piece1(row + 1)
                else:
                    piece2(row + 1)
            cp0.wait()
            cp1.wait()
            if t != nsteps - 1:
                clean()

    _call_cache.append(sc_call)
    return sc_call


def kernel(scalar, kernel_param, mask_param, b):
    n1, n2 = (jnp.asarray(a) for a in _noise())
    logits = jnp.tile(kernel_param, (B, 1))
    probs1 = jax.nn.softmax((logits + n1 / 1000) / TEMP, axis=1)
    logits_m = jnp.tile(mask_param, (B, 1))
    probs2 = jax.nn.softmax((logits_m + n2 / 1000) / TEMP, axis=1)
    s = jnp.maximum(jnp.float32(0.01), scalar)
    cval = (jnp.ones((1,), jnp.float32) / np.sqrt(N)) * s
    cvec = jnp.broadcast_to(cval, (L,))
    bits1 = lax.bitcast_convert_type(probs1, jnp.int32).reshape(-1)
    bits2 = lax.bitcast_convert_type(probs2, jnp.int32).reshape(-1)
    flat = _get_call()(bits1, bits2, cvec)
    return (flat.reshape(B, N // 8, M // 128, 8, 128)
            .transpose(0, 2, 4, 1, 3)
            .reshape(B, M, N))
